# baseline (speedup 1.0000x reference)
import functools

import jax
import jax.numpy as jnp
from jax import lax
from jax.experimental import pallas as pl
from jax.experimental.pallas import tpu as pltpu

N = 8192
D_MODEL = 2048
D_SAE = 16384
K = 64

EM = 512
EN = 1024
DM = 512
DK = 2048


def _encode_kernel(x_ref, w_ref, b_ref, z_ref):
    z = lax.dot_general(
        x_ref[...].astype(jnp.bfloat16), w_ref[...].astype(jnp.bfloat16),
        (((1,), (1,)), ((), ())),
        preferred_element_type=jnp.float32,
    )
    z_ref[...] = z + b_ref[...]


def _encode(x, W_enc, b_enc):
    m_blocks = N // EM
    n_blocks = D_SAE // EN
    return pl.pallas_call(
        _encode_kernel,
        grid=(m_blocks, n_blocks),
        in_specs=[
            pl.BlockSpec((EM, D_MODEL), lambda j, i: (j, 0)),
            pl.BlockSpec((EN, D_MODEL), lambda j, i: (i, 0)),
            pl.BlockSpec((1, EN), lambda j, i: (0, i)),
        ],
        out_specs=pl.BlockSpec((EM, EN), lambda j, i: (j, i)),
        out_shape=jax.ShapeDtypeStruct((N, D_SAE), jnp.float32),
    )(x, W_enc, b_enc.reshape(1, D_SAE))


def _decode_kernel(z_ref, w_ref, b_ref, out_ref):
    k = pl.program_id(1)
    zb = z_ref[...].astype(jnp.bfloat16)
    part = lax.dot_general(
        zb, w_ref[...],
        (((1,), (1,)), ((), ())),
        preferred_element_type=jnp.float32,
    )
    @pl.when(k == 0)
    def _():
        out_ref[...] = part + b_ref[...]

    @pl.when(k != 0)
    def _():
        out_ref[...] += part


def _decode(z_sparse, W_dec_bf, b_dec):
    m_blocks = N // DM
    k_blocks = D_SAE // DK
    return pl.pallas_call(
        _decode_kernel,
        grid=(m_blocks, k_blocks),
        in_specs=[
            pl.BlockSpec((DM, DK), lambda j, k: (j, k)),
            pl.BlockSpec((D_MODEL, DK), lambda j, k: (0, k)),
            pl.BlockSpec((1, D_MODEL), lambda j, k: (0, 0)),
        ],
        out_specs=pl.BlockSpec((DM, D_MODEL), lambda j, k: (j, 0)),
        out_shape=jax.ShapeDtypeStruct((N, D_MODEL), jnp.float32),
    )(z_sparse, W_dec_bf, b_dec.reshape(1, D_MODEL))


def kernel(x, W_enc, b_enc, W_dec, b_dec):
    z = _encode(x, W_enc, b_enc)
    topk_values, topk_indices = jax.lax.top_k(z, K)
    rows = jnp.arange(N)[:, None]
    z_sparse = jnp.zeros_like(z).at[rows, topk_indices].set(topk_values)
    W_dec_bf = W_dec.astype(jnp.bfloat16)
    x_hat = _decode(z_sparse, W_dec_bf, b_dec)
    return (x_hat, z_sparse)


# speedup vs baseline: 6.7223x; 6.7223x over previous
import functools

import jax
import jax.numpy as jnp
from jax import lax
from jax.experimental import pallas as pl
from jax.experimental.pallas import tpu as pltpu
from jax.experimental.pallas import tpu_sc as plsc

N = 8192
D_MODEL = 2048
D_SAE = 16384
K = 64

EM = 512
EN = 1024
CHUNK = 256
DM = 512
DK = 2048

NWORK = 32
ROWS_PER_W = N // NWORK
NVREG = D_SAE // 16
SURV_CAP = 2048


def _encode_kernel(x_ref, w_ref, b_ref, z_ref, c_ref, cmin_sc):
    i = pl.program_id(1)
    z = lax.dot_general(
        x_ref[...].astype(jnp.bfloat16), w_ref[...].astype(jnp.bfloat16),
        (((1,), (1,)), ((), ())),
        preferred_element_type=jnp.float32,
    )
    z = z + b_ref[...]
    z_ref[...] = z
    cm = jnp.max(z[:, 0:CHUNK], axis=1, keepdims=True)
    for k in range(1, EN // CHUNK):
        cm = jnp.minimum(cm, jnp.max(z[:, k * CHUNK:(k + 1) * CHUNK],
                                     axis=1, keepdims=True))

    @pl.when(i == 0)
    def _():
        cmin_sc[...] = cm

    @pl.when(i != 0)
    def _():
        cmin_sc[...] = jnp.minimum(cmin_sc[...], cm)

    @pl.when(i == pl.num_programs(1) - 1)
    def _():
        c_ref[...] = jnp.minimum(cmin_sc[...], cm)


def _encode(x, W_enc, b_enc):
    m_blocks = N // EM
    n_blocks = D_SAE // EN
    return pl.pallas_call(
        _encode_kernel,
        grid=(m_blocks, n_blocks),
        in_specs=[
            pl.BlockSpec((EM, D_MODEL), lambda j, i: (j, 0)),
            pl.BlockSpec((EN, D_MODEL), lambda j, i: (i, 0)),
            pl.BlockSpec((1, EN), lambda j, i: (0, i)),
        ],
        out_specs=[
            pl.BlockSpec((EM, EN), lambda j, i: (j, i)),
            pl.BlockSpec((EM, 1), lambda j, i: (j, 0)),
        ],
        out_shape=[
            jax.ShapeDtypeStruct((N, D_SAE), jnp.float32),
            jax.ShapeDtypeStruct((N, 1), jnp.float32),
        ],
        scratch_shapes=[pltpu.VMEM((EM, 1), jnp.float32)],
    )(x, W_enc, b_enc.reshape(1, D_SAE))


def _monotone_key(v):
    u = plsc.bitcast(v, jnp.int32)
    s = lax.shift_right_arithmetic(u, 31)
    m = lax.bitwise_or(s, jnp.int32(-2147483648))
    return plsc.bitcast(lax.bitwise_xor(u, m), jnp.uint32)


def _scalar_of(vec):
    return jnp.max(vec)


def _sc_topk_body(z_hbm, c_hbm, out_hbm, row_v, out_v, sidx, skey, c_v):
    wid = lax.axis_index("s") * 2 + lax.axis_index("c")
    base_row = wid * ROWS_PER_W

    lane = lax.iota(jnp.int32, 16)
    zero16f = jnp.zeros((16,), jnp.float32)
    zero16u = jnp.zeros((16,), jnp.uint32)
    zero16i = jnp.zeros((16,), jnp.int32)

    def zero_body(i, _):
        out_v[pl.ds(i * 16, 16)] = zero16f
        return 0
    lax.fori_loop(0, NVREG, zero_body, 0)

    pltpu.sync_copy(c_hbm.at[pl.ds(base_row, ROWS_PER_W)], c_v)

    def row_body(rr, _):
        r = base_row + rr
        pltpu.sync_copy(z_hbm.at[r], row_v)

        cchunk = c_v[pl.ds((rr // 16) * 16, 16)]
        csel = jnp.where(lane == (rr % 16).astype(jnp.int32), cchunk,
                         jnp.float32(-jnp.inf))
        c_scalar = _scalar_of(csel)
        c_b = jnp.full((16,), c_scalar, jnp.float32)

        def compact_body(i, off):
            v = row_v[pl.ds(i * 16, 16)]
            mask = v >= c_b
            ivec = lane + i * 16
            plsc.store_compressed(sidx.at[pl.ds(off, 16)], ivec, mask=mask)
            pc = _scalar_of(plsc.all_reduce_population_count(mask))
            off = jnp.minimum(off + pc, SURV_CAP - 16)
            return off
        n_s = lax.fori_loop(0, NVREG, compact_body, jnp.int32(0))

        sidx[pl.ds(n_s, 16)] = zero16i
        nv = (n_s + 15) // 16

        def key_body(i, _):
            iv = plsc.load_gather(row_v, [sidx[pl.ds(i * 16, 16)]])
            skey[pl.ds(i * 16, 16)] = _monotone_key(iv)
            return 0
        lax.fori_loop(0, nv, key_body, 0)
        skey[pl.ds(n_s, 16)] = zero16u

        def bit_body(b, prefix):
            bit = jnp.uint32(31) - b.astype(jnp.uint32)
            cand = prefix | lax.shift_left(jnp.uint32(1), bit)
            cand_b = jnp.full((16,), cand, jnp.uint32)

            def cnt_body(i, acc):
                kv = skey[pl.ds(i * 16, 16)]
                return acc + plsc.all_reduce_population_count(kv >= cand_b)
            cnt_vec = lax.fori_loop(0, nv, cnt_body,
                                    jnp.zeros((16,), jnp.int32))
            cnt = _scalar_of(cnt_vec)
            return jnp.where(cnt >= K, cand, prefix)
        tau = lax.fori_loop(0, 32, bit_body, jnp.uint32(0))
        tau_b = jnp.full((16,), tau, jnp.uint32)

        def scat_body(i, _):
            iv = sidx[pl.ds(i * 16, 16)]
            vv = plsc.load_gather(row_v, [iv])
            mask = _monotone_key(vv) >= tau_b
            plsc.store_scatter(out_v, [iv], vv, mask=mask)
            return 0
        lax.fori_loop(0, nv, scat_body, 0)

        pltpu.sync_copy(out_v, out_hbm.at[r])

        def unscat_body(i, _):
            iv = sidx[pl.ds(i * 16, 16)]
            vv = plsc.load_gather(out_v, [iv])
            mask = _monotone_key(vv) >= tau_b
            plsc.store_scatter(out_v, [iv], zero16f, mask=mask)
            return 0
        lax.fori_loop(0, nv, unscat_body, 0)
        return 0

    lax.fori_loop(0, ROWS_PER_W, row_body, 0)


def _sc_topk(z, c):
    mesh = plsc.VectorSubcoreMesh(core_axis_name="c", subcore_axis_name="s")
    kfn = pl.kernel(
        _sc_topk_body,
        mesh=mesh,
        compiler_params=pltpu.CompilerParams(needs_layout_passes=False),
        out_type=jax.ShapeDtypeStruct((N, D_SAE), jnp.float32),
        scratch_types=[
            pltpu.VMEM((D_SAE,), jnp.float32),
            pltpu.VMEM((D_SAE,), jnp.float32),
            pltpu.VMEM((SURV_CAP + 16,), jnp.int32),
            pltpu.VMEM((SURV_CAP + 32,), jnp.uint32),
            pltpu.VMEM((ROWS_PER_W,), jnp.float32),
        ],
    )
    return kfn(z, c)


def _decode_kernel(z_ref, w_ref, b_ref, out_ref):
    k = pl.program_id(1)
    zb = z_ref[...].astype(jnp.bfloat16)
    part = lax.dot_general(
        zb, w_ref[...],
        (((1,), (1,)), ((), ())),
        preferred_element_type=jnp.float32,
    )

    @pl.when(k == 0)
    def _():
        out_ref[...] = part + b_ref[...]

    @pl.when(k != 0)
    def _():
        out_ref[...] += part


def _decode(z_sparse, W_dec_bf, b_dec):
    m_blocks = N // DM
    k_blocks = D_SAE // DK
    return pl.pallas_call(
        _decode_kernel,
        grid=(m_blocks, k_blocks),
        in_specs=[
            pl.BlockSpec((DM, DK), lambda j, k: (j, k)),
            pl.BlockSpec((D_MODEL, DK), lambda j, k: (0, k)),
            pl.BlockSpec((1, D_MODEL), lambda j, k: (0, 0)),
        ],
        out_specs=pl.BlockSpec((DM, D_MODEL), lambda j, k: (j, 0)),
        out_shape=jax.ShapeDtypeStruct((N, D_MODEL), jnp.float32),
    )(z_sparse, W_dec_bf, b_dec.reshape(1, D_MODEL))


def kernel(x, W_enc, b_enc, W_dec, b_dec):
    z, c = _encode(x, W_enc, b_enc)
    z_sparse = _sc_topk(z, c.reshape(N))
    W_dec_bf = W_dec.astype(jnp.bfloat16)
    x_hat = _decode(z_sparse, W_dec_bf, b_dec)
    return (x_hat, z_sparse)


# speedup vs baseline: 8.1649x; 1.2146x over previous
import functools

import jax
import jax.numpy as jnp
from jax import lax
from jax.experimental import pallas as pl
from jax.experimental.pallas import tpu as pltpu
from jax.experimental.pallas import tpu_sc as plsc

N = 8192
D_MODEL = 2048
D_SAE = 16384
K = 64

EM = 512
EN = 1024
CHUNK = 256
DM = 512
DK = 2048

NWORK = 32
ROWS_PER_W = N // NWORK
NVREG = D_SAE // 16
SURV_CAP = 2048


def _encode_kernel(x_ref, w_ref, b_ref, z_ref, c_ref, cmin_sc):
    i = pl.program_id(1)
    z = lax.dot_general(
        x_ref[...].astype(jnp.bfloat16), w_ref[...].astype(jnp.bfloat16),
        (((1,), (1,)), ((), ())),
        preferred_element_type=jnp.float32,
    )
    z = z + b_ref[...]
    z_ref[...] = z
    cm = jnp.max(z[:, 0:CHUNK], axis=1, keepdims=True)
    for k in range(1, EN // CHUNK):
        cm = jnp.minimum(cm, jnp.max(z[:, k * CHUNK:(k + 1) * CHUNK],
                                     axis=1, keepdims=True))

    @pl.when(i == 0)
    def _():
        cmin_sc[...] = cm

    @pl.when(i != 0)
    def _():
        cmin_sc[...] = jnp.minimum(cmin_sc[...], cm)

    @pl.when(i == pl.num_programs(1) - 1)
    def _():
        c_ref[...] = jnp.minimum(cmin_sc[...], cm)


def _encode(x, W_enc_bf, b_enc):
    m_blocks = N // EM
    n_blocks = D_SAE // EN
    return pl.pallas_call(
        _encode_kernel,
        grid=(m_blocks, n_blocks),
        in_specs=[
            pl.BlockSpec((EM, D_MODEL), lambda j, i: (j, 0)),
            pl.BlockSpec((EN, D_MODEL), lambda j, i: (i, 0)),
            pl.BlockSpec((1, EN), lambda j, i: (0, i)),
        ],
        out_specs=[
            pl.BlockSpec((EM, EN), lambda j, i: (j, i)),
            pl.BlockSpec((EM, 1), lambda j, i: (j, 0)),
        ],
        out_shape=[
            jax.ShapeDtypeStruct((N, D_SAE), jnp.float32),
            jax.ShapeDtypeStruct((N, 1), jnp.float32),
        ],
        scratch_shapes=[pltpu.VMEM((EM, 1), jnp.float32)],
    )(x, W_enc_bf, b_enc.reshape(1, D_SAE))


def _monotone_key(v):
    u = plsc.bitcast(v, jnp.int32)
    s = lax.shift_right_arithmetic(u, 31)
    m = lax.bitwise_or(s, jnp.int32(-2147483648))
    return plsc.bitcast(lax.bitwise_xor(u, m), jnp.uint32)


def _inv_monotone_key(k):
    ki = plsc.bitcast(k, jnp.int32)
    nki = lax.bitwise_xor(ki, jnp.int32(-1))
    m = lax.bitwise_or(
        lax.shift_right_arithmetic(nki, 31),
        jnp.int32(-2147483648))
    return plsc.bitcast(lax.bitwise_xor(ki, m), jnp.float32)


def _sc_compact_body(z_hbm, c_hbm, keys_hbm, ns_hbm,
                     row0, row1, skey, c_v, ns_v, s_in0, s_in1):
    wid = lax.axis_index("s") * 2 + lax.axis_index("c")
    base_row = wid * ROWS_PER_W

    lane = lax.iota(jnp.int32, 16)
    zero16i = jnp.zeros((16,), jnp.int32)
    cap16 = jnp.full((16,), SURV_CAP - 16, jnp.int32)

    pltpu.sync_copy(c_hbm.at[pl.ds(base_row, ROWS_PER_W)], c_v)

    def process_row(rr, row_v):
        r = base_row + rr
        cchunk = c_v[pl.ds((rr // 16) * 16, 16)]
        csel = jnp.where(lane == (rr % 16).astype(jnp.int32), cchunk,
                         jnp.float32(-jnp.inf))
        c_b = jnp.full((16,), jnp.max(csel), jnp.float32)

        def compact_body(i, off):
            v = row_v[pl.ds(i * 16, 16)]
            mask = v >= c_b
            mi = mask.astype(jnp.int32)
            dest = off + plsc.cumsum(mi) - mi
            kv = plsc.bitcast(_monotone_key(v), jnp.int32)
            plsc.store_scatter(skey, [dest], kv, mask=mask)
            pc = plsc.all_reduce_population_count(mask)
            return jnp.minimum(off + pc, cap16)
        off_vec = lax.fori_loop(0, NVREG, compact_body, zero16i)
        ns_v[pl.ds(rr * 16, 16)] = off_vec
        pltpu.sync_copy(skey, keys_hbm.at[r])
        return 0

    PAIRS = ROWS_PER_W // 2
    pltpu.make_async_copy(z_hbm.at[base_row], row0, s_in0).start()

    def pair_body(i, _):
        r = base_row + 2 * i
        pltpu.make_async_copy(z_hbm.at[r + 1], row1, s_in1).start()
        pltpu.make_async_copy(z_hbm.at[r], row0, s_in0).wait()
        process_row(2 * i, row0)

        @pl.when(i < PAIRS - 1)
        def _():
            pltpu.make_async_copy(z_hbm.at[r + 2], row0, s_in0).start()
        pltpu.make_async_copy(z_hbm.at[r + 1], row1, s_in1).wait()
        process_row(2 * i + 1, row1)
        return 0

    lax.fori_loop(0, PAIRS, pair_body, 0)
    pltpu.sync_copy(ns_v, ns_hbm.at[wid])


def _sc_compact(z, c):
    mesh = plsc.VectorSubcoreMesh(core_axis_name="c", subcore_axis_name="s")
    kfn = pl.kernel(
        _sc_compact_body,
        mesh=mesh,
        compiler_params=pltpu.CompilerParams(needs_layout_passes=False),
        out_type=[
            jax.ShapeDtypeStruct((N, SURV_CAP), jnp.int32),
            jax.ShapeDtypeStruct((NWORK, ROWS_PER_W * 16), jnp.int32),
        ],
        scratch_types=[
            pltpu.VMEM((D_SAE,), jnp.float32),
            pltpu.VMEM((D_SAE,), jnp.float32),
            pltpu.VMEM((SURV_CAP,), jnp.int32),
            pltpu.VMEM((ROWS_PER_W,), jnp.float32),
            pltpu.VMEM((ROWS_PER_W * 16,), jnp.int32),
            pltpu.SemaphoreType.DMA,
            pltpu.SemaphoreType.DMA,
        ],
    )
    return kfn(z, c)


BM = 512


def _tc_binsearch_kernel(keys_ref, ns_ref, tau_ref):
    m = jnp.int32(-2147483648)
    keys = keys_ref[...]
    ks = lax.bitwise_xor(keys, m)
    col = lax.broadcasted_iota(jnp.int32, (BM, SURV_CAP), 1)
    valid = col < ns_ref[...]
    kvec = jnp.full((BM, 1), K, jnp.int32)
    prefix = jnp.zeros((BM, 1), jnp.int32)
    for b in range(32):
        bit = jnp.int32(-2147483648) if b == 0 else jnp.int32(1 << (31 - b))
        cand = lax.bitwise_or(prefix, bit)
        hits = (ks >= lax.bitwise_xor(cand, m)) & valid
        cnt = jnp.sum(hits.astype(jnp.int32), axis=1, keepdims=True)
        prefix = jnp.where(cnt >= kvec, cand, prefix)
    nki = lax.bitwise_xor(prefix, jnp.int32(-1))
    mm = lax.bitwise_or(lax.shift_right_arithmetic(nki, 31), m)
    tau_ref[...] = lax.bitcast_convert_type(
        lax.bitwise_xor(prefix, mm), jnp.float32)


def _tc_binsearch(keys, ns):
    m_blocks = N // BM
    return pl.pallas_call(
        _tc_binsearch_kernel,
        grid=(m_blocks,),
        in_specs=[
            pl.BlockSpec((BM, SURV_CAP), lambda j: (j, 0)),
            pl.BlockSpec((BM, 1), lambda j: (j, 0)),
        ],
        out_specs=pl.BlockSpec((BM, 1), lambda j: (j, 0)),
        out_shape=jax.ShapeDtypeStruct((N, 1), jnp.float32),
    )(keys, ns)


def _decode_kernel(z_ref, tau_ref, w_ref, b_ref, out_ref, zs_ref):
    k = pl.program_id(1)
    z = z_ref[...]
    zs = jnp.where(z >= tau_ref[...], z, jnp.float32(0))
    zs_ref[...] = zs
    part = lax.dot_general(
        zs.astype(jnp.bfloat16), w_ref[...],
        (((1,), (1,)), ((), ())),
        preferred_element_type=jnp.float32,
    )

    @pl.when(k == 0)
    def _():
        out_ref[...] = part + b_ref[...]

    @pl.when(k != 0)
    def _():
        out_ref[...] += part


def _decode(z, tau, W_dec_bf, b_dec):
    m_blocks = N // DM
    k_blocks = D_SAE // DK
    return pl.pallas_call(
        _decode_kernel,
        grid=(m_blocks, k_blocks),
        in_specs=[
            pl.BlockSpec((DM, DK), lambda j, k: (j, k)),
            pl.BlockSpec((DM, 1), lambda j, k: (j, 0)),
            pl.BlockSpec((D_MODEL, DK), lambda j, k: (0, k)),
            pl.BlockSpec((1, D_MODEL), lambda j, k: (0, 0)),
        ],
        out_specs=[
            pl.BlockSpec((DM, D_MODEL), lambda j, k: (j, 0)),
            pl.BlockSpec((DM, DK), lambda j, k: (j, k)),
        ],
        out_shape=[
            jax.ShapeDtypeStruct((N, D_MODEL), jnp.float32),
            jax.ShapeDtypeStruct((N, D_SAE), jnp.float32),
        ],
    )(z, tau, W_dec_bf, b_dec.reshape(1, D_MODEL))


def kernel(x, W_enc, b_enc, W_dec, b_dec):
    z, c = _encode(x, W_enc.astype(jnp.bfloat16), b_enc)
    keys, ns = _sc_compact(z, c.reshape(N))
    tau = _tc_binsearch(keys, ns.reshape(N, 16)[:, :1])
    W_dec_bf = W_dec.astype(jnp.bfloat16)
    x_hat, z_sparse = _decode(z, tau, W_dec_bf, b_dec)
    return (x_hat, z_sparse)
